# Initial kernel scaffold; baseline (speedup 1.0000x reference)
import jax
import jax.numpy as jnp
from jax.experimental import pallas as pl


def kernel(x, edge_index, W1, b1, W2, b2):
    raise NotImplementedError("write your pallas kernel here")


# speedup vs baseline: 16.0601x; 16.0601x over previous
import functools

import jax
import jax.numpy as jnp
from jax import lax
from jax.experimental import pallas as pl
from jax.experimental.pallas import tpu as pltpu
from jax.experimental.pallas import tpu_sc as plsc

N = 10000
E = 320000
D = 128

NC = 2
NS = 16
NW = NC * NS
EPW = E // NW
K = 80
NCHUNK = EPW // K
ZT = 10
ZR = N // ZT
OC = 200
NOC = ZR // OC


def _make_deg_kernel(mesh):
    @functools.partial(
        pl.kernel,
        out_type=jax.ShapeDtypeStruct((NC * N,), jnp.float32),
        mesh=mesh,
        scratch_types=[
            pltpu.VMEM((K,), jnp.int32),
            pltpu.VMEM((K,), jnp.int32),
            pltpu.VMEM((K,), jnp.float32),
            pltpu.VMEM((1000,), jnp.float32),
            pltpu.SemaphoreType.DMA,
            pltpu.SemaphoreType.DMA,
            pltpu.VMEM_SHARED((N,), jnp.float32),
        ],
    )
    def deg_kernel(col_hbm, zeros_hbm, out_hbm, c0, c1, ones_v, buf_v, s0, s1, acc):
        c = lax.axis_index("c")
        s = lax.axis_index("s")
        base = (c * NS + s) * EPW

        @pl.when(s == 0)
        def _():
            pltpu.sync_copy(zeros_hbm, acc)

        for j in range(K // 16):
            ones_v[pl.ds(j * 16, 16)] = jnp.ones((16,), jnp.float32)
        plsc.subcore_barrier()

        cidx = (c0, c1)
        sem = (s0, s1)

        def start_idx(i, b):
            pltpu.async_copy(col_hbm.at[pl.ds(base + i * K, K)], cidx[b], sem[b])

        def wait_idx(i, b):
            pltpu.make_async_copy(
                col_hbm.at[pl.ds(base + i * K, K)], cidx[b], sem[b]).wait()

        def step(i, b):
            wait_idx(i, b)

            @pl.when(i + 1 < NCHUNK)
            def _():
                start_idx(i + 1, 1 - b)

            pltpu.sync_copy(ones_v, acc.at[cidx[b]], add=True)

        start_idx(0, 0)

        def body(g, carry):
            step(2 * g, 0)
            step(2 * g + 1, 1)
            return carry

        lax.fori_loop(0, NCHUNK // 2, body, 0)
        step(NCHUNK - 1, 0)

        plsc.subcore_barrier()

        @pl.when(s < ZT)
        def _():
            pltpu.sync_copy(acc.at[pl.ds(s * ZR, ZR)], buf_v)
            pltpu.sync_copy(buf_v, out_hbm.at[pl.ds(c * N + s * ZR, ZR)])

    return deg_kernel


def _make_seg_kernel(mesh):
    @functools.partial(
        pl.kernel,
        out_type=jax.ShapeDtypeStruct((NC, N, D), jnp.float32),
        mesh=mesh,
        scratch_types=[
            pltpu.VMEM((K,), jnp.int32),
            pltpu.VMEM((K,), jnp.int32),
            pltpu.VMEM((K,), jnp.int32),
            pltpu.VMEM((K,), jnp.int32),
            pltpu.VMEM((K, D), jnp.float32),
            pltpu.VMEM((K, D), jnp.float32),
            pltpu.VMEM((OC, D), jnp.float32),
            pltpu.SemaphoreType.DMA,
            pltpu.SemaphoreType.DMA,
            pltpu.SemaphoreType.DMA,
            pltpu.SemaphoreType.DMA,
            pltpu.SemaphoreType.DMA,
            pltpu.SemaphoreType.DMA,
            pltpu.VMEM_SHARED((N, D), jnp.float32),
        ],
    )
    def seg_kernel(g_hbm, row_hbm, col_hbm, zeros_hbm, out_hbm,
                   r0, r1, c0, c1, w0, w1, ob,
                   sr0, sr1, sc0, sc1, sg0, sg1, acc):
        c = lax.axis_index("c")
        s = lax.axis_index("s")
        base = (c * NS + s) * EPW

        @pl.when(s < ZT)
        def _():
            pltpu.sync_copy(zeros_hbm, acc.at[pl.ds(s * ZR, ZR)])

        plsc.subcore_barrier()

        ridx = (r0, r1)
        cidx = (c0, c1)
        rows = (w0, w1)
        sr = (sr0, sr1)
        sc = (sc0, sc1)
        sg = (sg0, sg1)

        def start_idx(i, b):
            pltpu.async_copy(row_hbm.at[pl.ds(base + i * K, K)], ridx[b], sr[b])
            pltpu.async_copy(col_hbm.at[pl.ds(base + i * K, K)], cidx[b], sc[b])

        def wait_idx(i, b):
            pltpu.make_async_copy(
                row_hbm.at[pl.ds(base + i * K, K)], ridx[b], sr[b]).wait()
            pltpu.make_async_copy(
                col_hbm.at[pl.ds(base + i * K, K)], cidx[b], sc[b]).wait()

        def step(i, b):
            wait_idx(i, b)
            pltpu.async_copy(g_hbm.at[ridx[b]], rows[b], sg[b])
            pltpu.make_async_copy(
                g_hbm.at[ridx[1 - b]], rows[1 - b], sg[1 - b]).wait()
            pltpu.sync_copy(rows[1 - b], acc.at[cidx[1 - b]], add=True)

            @pl.when(i + 1 < NCHUNK)
            def _():
                start_idx(i + 1, 1 - b)

        start_idx(0, 0)
        wait_idx(0, 0)
        pltpu.async_copy(g_hbm.at[ridx[0]], rows[0], sg[0])
        start_idx(1, 1)

        def body(g, carry):
            step(2 * g + 1, 1)
            step(2 * g + 2, 0)
            return carry

        lax.fori_loop(0, (NCHUNK - 1) // 2, body, 0)

        pltpu.make_async_copy(g_hbm.at[ridx[0]], rows[0], sg[0]).wait()
        pltpu.sync_copy(rows[0], acc.at[cidx[0]], add=True)

        plsc.subcore_barrier()

        @pl.when(s < ZT)
        def _():
            for j in range(NOC):
                off = s * ZR + j * OC
                pltpu.sync_copy(acc.at[pl.ds(off, OC)], ob)
                pltpu.sync_copy(ob, out_hbm.at[c, pl.ds(off, OC)])

    return seg_kernel


@functools.cache
def _sc_kernels():
    mesh = plsc.VectorSubcoreMesh(core_axis_name="c", subcore_axis_name="s")
    return _make_deg_kernel(mesh), _make_seg_kernel(mesh)


RB = 400
NB = N // RB

_tc_params = pltpu.CompilerParams(dimension_semantics=("parallel",))


def _g1_body(d0_ref, d1_ref, x_ref, w_ref, g1_ref, dis_ref):
    dis = lax.rsqrt(d0_ref[...] + d1_ref[...] + 1.0)
    g1_ref[...] = dis * jnp.dot(x_ref[...], w_ref[...],
                                preferred_element_type=jnp.float32)
    dis_ref[...] = dis


_g1_call = pl.pallas_call(
    _g1_body,
    grid=(NB,),
    in_specs=[
        pl.BlockSpec((RB, 1), lambda i: (i, 0)),
        pl.BlockSpec((RB, 1), lambda i: (i, 0)),
        pl.BlockSpec((RB, D), lambda i: (i, 0)),
        pl.BlockSpec((D, D), lambda i: (0, 0)),
    ],
    out_specs=[
        pl.BlockSpec((RB, D), lambda i: (i, 0)),
        pl.BlockSpec((RB, 1), lambda i: (i, 0)),
    ],
    out_shape=[
        jax.ShapeDtypeStruct((N, D), jnp.float32),
        jax.ShapeDtypeStruct((N, 1), jnp.float32),
    ],
    compiler_params=_tc_params,
)


def _l2_body(a0_ref, a1_ref, g1_ref, dis_ref, b1_ref, w2_ref, g2_ref):
    pre = dis_ref[...] * (a0_ref[...] + a1_ref[...] + g1_ref[...]) + b1_ref[...]
    act = jnp.where(pre >= 0, pre, 0.01 * pre)
    g2_ref[...] = dis_ref[...] * jnp.dot(act, w2_ref[...],
                                         preferred_element_type=jnp.float32)


_l2_call = pl.pallas_call(
    _l2_body,
    grid=(NB,),
    in_specs=[
        pl.BlockSpec((RB, D), lambda i: (i, 0)),
        pl.BlockSpec((RB, D), lambda i: (i, 0)),
        pl.BlockSpec((RB, D), lambda i: (i, 0)),
        pl.BlockSpec((RB, 1), lambda i: (i, 0)),
        pl.BlockSpec((1, D), lambda i: (0, 0)),
        pl.BlockSpec((D, D), lambda i: (0, 0)),
    ],
    out_specs=pl.BlockSpec((RB, D), lambda i: (i, 0)),
    out_shape=jax.ShapeDtypeStruct((N, D), jnp.float32),
    compiler_params=_tc_params,
)


def _act_body(a0_ref, a1_ref, g2_ref, dis_ref, b2_ref, act_ref):
    pre = dis_ref[...] * (a0_ref[...] + a1_ref[...] + g2_ref[...]) + b2_ref[...]
    act_ref[...] = jnp.where(pre >= 0, pre, 0.01 * pre)


_act_call = pl.pallas_call(
    _act_body,
    grid=(NB,),
    in_specs=[
        pl.BlockSpec((RB, D), lambda i: (i, 0)),
        pl.BlockSpec((RB, D), lambda i: (i, 0)),
        pl.BlockSpec((RB, D), lambda i: (i, 0)),
        pl.BlockSpec((RB, 1), lambda i: (i, 0)),
        pl.BlockSpec((1, D), lambda i: (0, 0)),
    ],
    out_specs=pl.BlockSpec((RB, D), lambda i: (i, 0)),
    out_shape=jax.ShapeDtypeStruct((N, D), jnp.float32),
    compiler_params=_tc_params,
)


BM = 1024
BN = 2048


def _sim_body(a_ref, b_ref, o_ref):
    o_ref[...] = jax.nn.sigmoid(
        lax.dot_general(a_ref[...], b_ref[...], (((1,), (1,)), ((), ())),
                        preferred_element_type=jnp.float32))


_sim_call = pl.pallas_call(
    _sim_body,
    grid=(pl.cdiv(N, BM), pl.cdiv(N, BN)),
    in_specs=[
        pl.BlockSpec((BM, D), lambda i, j: (i, 0)),
        pl.BlockSpec((BN, D), lambda i, j: (j, 0)),
    ],
    out_specs=pl.BlockSpec((BM, BN), lambda i, j: (i, j)),
    out_shape=jax.ShapeDtypeStruct((N, N), jnp.float32),
    compiler_params=pltpu.CompilerParams(
        dimension_semantics=("parallel", "parallel")),
)


def kernel(x, edge_index, W1, b1, W2, b2):
    deg_kernel, seg_kernel = _sc_kernels()
    row = edge_index[0]
    col = edge_index[1]
    zeros_deg = jnp.zeros((N,), jnp.float32)
    zeros_seg = jnp.zeros((ZR, D), jnp.float32)

    degp = deg_kernel(col, zeros_deg).reshape(NC, N)
    d0 = degp[0].reshape(N, 1)
    d1 = degp[1].reshape(N, 1)
    g1, dis = _g1_call(d0, d1, x, W1)
    accp1 = seg_kernel(g1, row, col, zeros_seg)
    g2 = _l2_call(accp1[0], accp1[1], g1, dis, b1.reshape(1, D), W2)
    accp2 = seg_kernel(g2, row, col, zeros_seg)
    act2 = _act_call(accp2[0], accp2[1], g2, dis, b2.reshape(1, D))
    return _sim_call(act2, act2)


# speedup vs baseline: 20.0144x; 1.2462x over previous
import functools

import jax
import jax.numpy as jnp
from jax import lax
from jax.experimental import pallas as pl
from jax.experimental.pallas import tpu as pltpu
from jax.experimental.pallas import tpu_sc as plsc

N = 10000
E = 320000
D = 128

NC = 2
NS = 16
NW = NC * NS
EPW = E // NW
K = 80
NCHUNK = EPW // K
ZT = 10
ZR = N // ZT
OC = 200
NOC = ZR // OC


RING = 4


def _make_deg_kernel(mesh):
    @functools.partial(
        pl.kernel,
        out_type=jax.ShapeDtypeStruct((NC * N,), jnp.float32),
        mesh=mesh,
        scratch_types=[
            [pltpu.VMEM((K,), jnp.int32)] * RING,
            pltpu.VMEM((K,), jnp.float32),
            pltpu.VMEM((ZR,), jnp.float32),
            [pltpu.SemaphoreType.DMA] * RING,
            [pltpu.SemaphoreType.DMA] * RING,
            pltpu.VMEM_SHARED((N,), jnp.float32),
        ],
    )
    def deg_kernel(col_hbm, zeros_hbm, out_hbm, cidx, ones_v, buf_v, si, ss, acc):
        c = lax.axis_index("c")
        s = lax.axis_index("s")
        base = (c * NS + s) * EPW

        def start_idx(i, b):
            pltpu.async_copy(col_hbm.at[pl.ds(base + i * K, K)], cidx[b], si[b])

        def wait_idx(i, b):
            pltpu.make_async_copy(
                col_hbm.at[pl.ds(base + i * K, K)], cidx[b], si[b]).wait()

        def wait_sc(b):
            pltpu.make_async_copy(ones_v, acc.at[cidx[b]], ss[b]).wait()

        start_idx(0, 0)
        start_idx(1, 1)

        @pl.when(s == 0)
        def _():
            pltpu.sync_copy(zeros_hbm, acc)

        for j in range(K // 16):
            ones_v[pl.ds(j * 16, 16)] = jnp.ones((16,), jnp.float32)
        plsc.subcore_barrier()

        def step(i, b, drain=True, prefetch=True):
            wait_idx(i, b)
            pltpu.async_copy(ones_v, acc.at[cidx[b]], ss[b], add=True)
            if drain:
                wait_sc((b + 2) % RING)
            if prefetch:
                start_idx(i + 2, (b + 2) % RING)

        step(0, 0, drain=False)
        step(1, 1, drain=False)

        def body(g, carry):
            for r in range(RING):
                step(4 * g + 2 + r, (2 + r) % RING)
            return carry

        lax.fori_loop(0, (NCHUNK - 3) // RING, body, 0)
        step(122, 2)
        step(123, 3, prefetch=False)
        step(124, 0, prefetch=False)
        wait_sc(3)
        wait_sc(0)

        plsc.subcore_barrier()

        @pl.when(s < ZT)
        def _():
            pltpu.sync_copy(acc.at[pl.ds(s * ZR, ZR)], buf_v)
            pltpu.sync_copy(buf_v, out_hbm.at[pl.ds(c * N + s * ZR, ZR)])

    return deg_kernel


def _make_seg_kernel(mesh):
    @functools.partial(
        pl.kernel,
        out_type=jax.ShapeDtypeStruct((NC, N, D), jnp.float32),
        mesh=mesh,
        scratch_types=[
            [pltpu.VMEM((K,), jnp.int32)] * RING,
            [pltpu.VMEM((K,), jnp.int32)] * RING,
            [pltpu.VMEM((K, D), jnp.float32)] * RING,
            [pltpu.SemaphoreType.DMA] * RING,
            [pltpu.SemaphoreType.DMA] * RING,
            [pltpu.SemaphoreType.DMA] * RING,
            [pltpu.SemaphoreType.DMA] * RING,
            pltpu.VMEM_SHARED((N, D), jnp.float32),
        ],
    )
    def seg_kernel(g_hbm, row_hbm, col_hbm, zeros_hbm, out_hbm,
                   ridx, cidx, rows, sr, sc, sg, ss, acc):
        c = lax.axis_index("c")
        s = lax.axis_index("s")
        base = (c * NS + s) * EPW

        def start_idx(i, b):
            pltpu.async_copy(row_hbm.at[pl.ds(base + i * K, K)], ridx[b], sr[b])
            pltpu.async_copy(col_hbm.at[pl.ds(base + i * K, K)], cidx[b], sc[b])

        def wait_idx(i, b):
            pltpu.make_async_copy(
                row_hbm.at[pl.ds(base + i * K, K)], ridx[b], sr[b]).wait()
            pltpu.make_async_copy(
                col_hbm.at[pl.ds(base + i * K, K)], cidx[b], sc[b]).wait()

        def wait_gather(b):
            pltpu.make_async_copy(g_hbm.at[ridx[b]], rows[b], sg[b]).wait()

        def wait_sc(b):
            pltpu.make_async_copy(rows[b], acc.at[cidx[b]], ss[b]).wait()

        start_idx(0, 0)
        start_idx(1, 1)

        @pl.when(s < ZT)
        def _():
            pltpu.sync_copy(zeros_hbm, acc.at[pl.ds(s * ZR, ZR)])

        plsc.subcore_barrier()

        def step(i, b, drain=True, prefetch=True):
            wait_idx(i, b)
            pltpu.async_copy(g_hbm.at[ridx[b]], rows[b], sg[b])
            bp = (b + RING - 1) % RING
            wait_gather(bp)
            pltpu.async_copy(rows[bp], acc.at[cidx[bp]], ss[bp], add=True)
            if drain:
                wait_sc((b + 2) % RING)
            if prefetch:
                start_idx(i + 2, (b + 2) % RING)

        wait_idx(0, 0)
        pltpu.async_copy(g_hbm.at[ridx[0]], rows[0], sg[0])
        start_idx(2, 2)
        step(1, 1, drain=False)

        def body(g, carry):
            for r in range(RING):
                step(4 * g + 2 + r, (2 + r) % RING)
            return carry

        lax.fori_loop(0, (NCHUNK - 3) // RING, body, 0)
        step(122, 2)
        step(123, 3, prefetch=False)
        step(124, 0, prefetch=False)
        wait_gather(0)
        pltpu.async_copy(rows[0], acc.at[cidx[0]], ss[0], add=True)
        wait_sc(3)
        wait_sc(0)

        plsc.subcore_barrier()

        @pl.when(s < ZT)
        def _():
            nfull = ZR // K
            tail = ZR - nfull * K
            for j in range(nfull + 1):
                b = j % RING
                off = s * ZR + j * K
                n = K if j < nfull else tail
                if j >= RING:
                    poff = s * ZR + (j - RING) * K
                    pltpu.make_async_copy(
                        rows[b], out_hbm.at[c, pl.ds(poff, K)], ss[b]).wait()
                pltpu.sync_copy(acc.at[pl.ds(off, n)], rows[b].at[pl.ds(0, n)])
                pltpu.async_copy(rows[b].at[pl.ds(0, n)],
                                 out_hbm.at[c, pl.ds(off, n)], ss[b])
            for j in range(nfull + 1 - RING, nfull + 1):
                b = j % RING
                off = s * ZR + j * K
                n = K if j < nfull else tail
                pltpu.make_async_copy(
                    rows[b].at[pl.ds(0, n)],
                    out_hbm.at[c, pl.ds(off, n)], ss[b]).wait()

    return seg_kernel


@functools.cache
def _sc_kernels():
    mesh = plsc.VectorSubcoreMesh(core_axis_name="c", subcore_axis_name="s")
    return _make_deg_kernel(mesh), _make_seg_kernel(mesh)


RB = 400
NB = N // RB

_tc_params = pltpu.CompilerParams(dimension_semantics=("parallel",))


def _g1_body(d0_ref, d1_ref, x_ref, w_ref, g1_ref, dis_ref):
    dis = lax.rsqrt(d0_ref[...] + d1_ref[...] + 1.0)
    g1_ref[...] = dis * jnp.dot(x_ref[...], w_ref[...],
                                preferred_element_type=jnp.float32)
    dis_ref[...] = dis


_g1_call = pl.pallas_call(
    _g1_body,
    grid=(NB,),
    in_specs=[
        pl.BlockSpec((RB, 1), lambda i: (i, 0)),
        pl.BlockSpec((RB, 1), lambda i: (i, 0)),
        pl.BlockSpec((RB, D), lambda i: (i, 0)),
        pl.BlockSpec((D, D), lambda i: (0, 0)),
    ],
    out_specs=[
        pl.BlockSpec((RB, D), lambda i: (i, 0)),
        pl.BlockSpec((RB, 1), lambda i: (i, 0)),
    ],
    out_shape=[
        jax.ShapeDtypeStruct((N, D), jnp.float32),
        jax.ShapeDtypeStruct((N, 1), jnp.float32),
    ],
    compiler_params=_tc_params,
)


def _l2_body(a0_ref, a1_ref, g1_ref, dis_ref, b1_ref, w2_ref, g2_ref):
    pre = dis_ref[...] * (a0_ref[...] + a1_ref[...] + g1_ref[...]) + b1_ref[...]
    act = jnp.where(pre >= 0, pre, 0.01 * pre)
    g2_ref[...] = dis_ref[...] * jnp.dot(act, w2_ref[...],
                                         preferred_element_type=jnp.float32)


_l2_call = pl.pallas_call(
    _l2_body,
    grid=(NB,),
    in_specs=[
        pl.BlockSpec((RB, D), lambda i: (i, 0)),
        pl.BlockSpec((RB, D), lambda i: (i, 0)),
        pl.BlockSpec((RB, D), lambda i: (i, 0)),
        pl.BlockSpec((RB, 1), lambda i: (i, 0)),
        pl.BlockSpec((1, D), lambda i: (0, 0)),
        pl.BlockSpec((D, D), lambda i: (0, 0)),
    ],
    out_specs=pl.BlockSpec((RB, D), lambda i: (i, 0)),
    out_shape=jax.ShapeDtypeStruct((N, D), jnp.float32),
    compiler_params=_tc_params,
)


def _act_body(a0_ref, a1_ref, g2_ref, dis_ref, b2_ref, act_ref):
    pre = dis_ref[...] * (a0_ref[...] + a1_ref[...] + g2_ref[...]) + b2_ref[...]
    act_ref[...] = jnp.where(pre >= 0, pre, 0.01 * pre)


_act_call = pl.pallas_call(
    _act_body,
    grid=(NB,),
    in_specs=[
        pl.BlockSpec((RB, D), lambda i: (i, 0)),
        pl.BlockSpec((RB, D), lambda i: (i, 0)),
        pl.BlockSpec((RB, D), lambda i: (i, 0)),
        pl.BlockSpec((RB, 1), lambda i: (i, 0)),
        pl.BlockSpec((1, D), lambda i: (0, 0)),
    ],
    out_specs=pl.BlockSpec((RB, D), lambda i: (i, 0)),
    out_shape=jax.ShapeDtypeStruct((N, D), jnp.float32),
    compiler_params=_tc_params,
)


BM = 1024
BN = 2048


def _sim_body(a_ref, b_ref, o_ref):
    o_ref[...] = jax.nn.sigmoid(
        lax.dot_general(a_ref[...], b_ref[...], (((1,), (1,)), ((), ())),
                        preferred_element_type=jnp.float32))


_sim_call = pl.pallas_call(
    _sim_body,
    grid=(pl.cdiv(N, BM), pl.cdiv(N, BN)),
    in_specs=[
        pl.BlockSpec((BM, D), lambda i, j: (i, 0)),
        pl.BlockSpec((BN, D), lambda i, j: (j, 0)),
    ],
    out_specs=pl.BlockSpec((BM, BN), lambda i, j: (i, j)),
    out_shape=jax.ShapeDtypeStruct((N, N), jnp.float32),
    compiler_params=pltpu.CompilerParams(
        dimension_semantics=("parallel", "parallel")),
)


def kernel(x, edge_index, W1, b1, W2, b2):
    deg_kernel, seg_kernel = _sc_kernels()
    row = edge_index[0]
    col = edge_index[1]
    zeros_deg = jnp.zeros((N,), jnp.float32)
    zeros_seg = jnp.zeros((ZR, D), jnp.float32)

    degp = deg_kernel(col, zeros_deg).reshape(NC, N)
    d0 = degp[0].reshape(N, 1)
    d1 = degp[1].reshape(N, 1)
    g1, dis = _g1_call(d0, d1, x, W1)
    accp1 = seg_kernel(g1, row, col, zeros_seg)
    g2 = _l2_call(accp1[0], accp1[1], g1, dis, b1.reshape(1, D), W2)
    accp2 = seg_kernel(g2, row, col, zeros_seg)
    act2 = _act_call(accp2[0], accp2[1], g2, dis, b2.reshape(1, D))
    return _sim_call(act2, act2)
